# baseline (speedup 1.0000x reference)
import functools

import jax
import jax.numpy as jnp
from jax import lax
from jax.experimental import pallas as pl
from jax.experimental.pallas import tpu as pltpu
from jax.experimental.pallas import tpu_sc as plsc

NUM_ROWS = 100000
EMBED_DIM = 64
BATCH = 16384

_FEATS_PER_SUB = EMBED_DIM // 16
_HALF = BATCH // 2


_CHUNK = 2048
_NCHUNK = BATCH // _CHUNK


def _gather_table(tref, idxref, oref, fb, idx_v, row_v, out_a, out_b, sems):
    cp_idx = pltpu.async_copy(idxref, idx_v, sems[2])
    cp_row = pltpu.async_copy(tref.at[fb * _FEATS_PER_SUB], row_v, sems[3])
    cp_idx.wait()
    cp_row.wait()
    bufs = (out_a, out_b)
    pending = [None, None]
    seg = 0
    for j in range(_FEATS_PER_SUB):
        d = fb * _FEATS_PER_SUB + j
        if j > 0:
            pltpu.sync_copy(tref.at[d], row_v)
        for q in range(_NCHUNK):
            b = seg % 2
            if pending[b] is not None:
                pending[b].wait()
            buf = bufs[b]

            @plsc.parallel_loop(0, _CHUNK, step=16, unroll=8)
            def _(i, q=q, buf=buf):
                ids = idx_v[pl.ds(q * _CHUNK + i, 16)]
                buf[pl.ds(i, 16)] = plsc.load_gather(row_v, [ids])

            pending[b] = pltpu.async_copy(
                buf, oref.at[d, pl.ds(q * _CHUNK, _CHUNK)], sems[b])
            seg += 1
    pending[0].wait()
    pending[1].wait()


def _emb_kernel(ut, it, uid_hbm, iid_hbm, out_u, out_i,
                idx_v, row_v, out_a, out_b, s0, s1, s2, s3):
    core = lax.axis_index("c")
    fb = lax.axis_index("s")
    sems = (s0, s1, s2, s3)

    @pl.when(core == 0)
    def _():
        _gather_table(ut, uid_hbm, out_u, fb, idx_v, row_v, out_a, out_b, sems)

    @pl.when(core == 1)
    def _():
        _gather_table(it, iid_hbm, out_i, fb, idx_v, row_v, out_a, out_b, sems)


@jax.jit
def kernel(user_embedding, item_embedding, user_ids, item_ids):
    mesh = plsc.VectorSubcoreMesh(core_axis_name="c", subcore_axis_name="s")
    fn = functools.partial(
        pl.kernel,
        mesh=mesh,
        compiler_params=pltpu.CompilerParams(needs_layout_passes=False),
        out_type=(
            jax.ShapeDtypeStruct((EMBED_DIM, BATCH), jnp.float32),
            jax.ShapeDtypeStruct((EMBED_DIM, BATCH), jnp.float32),
        ),
        scratch_types=[
            pltpu.VMEM((BATCH,), jnp.int32),
            pltpu.VMEM((NUM_ROWS,), jnp.float32),
            pltpu.VMEM((_CHUNK,), jnp.float32),
            pltpu.VMEM((_CHUNK,), jnp.float32),
            pltpu.SemaphoreType.DMA,
            pltpu.SemaphoreType.DMA,
            pltpu.SemaphoreType.DMA,
            pltpu.SemaphoreType.DMA,
        ],
    )(_emb_kernel)
    out_ut, out_it = fn(user_embedding.T, item_embedding.T,
                        user_ids.astype(jnp.int32), item_ids.astype(jnp.int32))
    return (out_ut.T, out_it.T)


# speedup vs baseline: 1.0555x; 1.0555x over previous
import functools

import jax
import jax.numpy as jnp
from jax import lax
from jax.experimental import pallas as pl
from jax.experimental.pallas import tpu as pltpu
from jax.experimental.pallas import tpu_sc as plsc

NUM_ROWS = 100000
EMBED_DIM = 64
BATCH = 16384

_FEATS_PER_SUB = EMBED_DIM // 16
_HALF = BATCH // 2


_CHUNK = 4096
_NCHUNK = BATCH // _CHUNK


def _gather_table(tref, idxref, oref, fb, idx_v, row_v, out_a, out_b, sems):
    cp_idx = pltpu.async_copy(idxref, idx_v, sems[2])
    cp_row = pltpu.async_copy(tref.at[fb * _FEATS_PER_SUB], row_v, sems[3])
    cp_idx.wait()
    cp_row.wait()
    bufs = (out_a, out_b)
    pending = [None, None]
    seg = 0
    for j in range(_FEATS_PER_SUB):
        d = fb * _FEATS_PER_SUB + j
        if j > 0:
            pltpu.sync_copy(tref.at[d], row_v)
        for q in range(_NCHUNK):
            b = seg % 2
            if pending[b] is not None:
                pending[b].wait()
            buf = bufs[b]

            @plsc.parallel_loop(0, _CHUNK, step=16, unroll=4)
            def _(i, q=q, buf=buf):
                ids = idx_v[pl.ds(q * _CHUNK + i, 16)]
                buf[pl.ds(i, 16)] = plsc.load_gather(row_v, [ids])

            pending[b] = pltpu.async_copy(
                buf, oref.at[d, pl.ds(q * _CHUNK, _CHUNK)], sems[b])
            seg += 1
    pending[0].wait()
    pending[1].wait()


def _emb_kernel(ut, it, uid_hbm, iid_hbm, out_u, out_i,
                idx_v, row_v, out_a, out_b, s0, s1, s2, s3):
    core = lax.axis_index("c")
    fb = lax.axis_index("s")
    sems = (s0, s1, s2, s3)

    @pl.when(core == 0)
    def _():
        _gather_table(ut, uid_hbm, out_u, fb, idx_v, row_v, out_a, out_b, sems)

    @pl.when(core == 1)
    def _():
        _gather_table(it, iid_hbm, out_i, fb, idx_v, row_v, out_a, out_b, sems)


@jax.jit
def kernel(user_embedding, item_embedding, user_ids, item_ids):
    mesh = plsc.VectorSubcoreMesh(core_axis_name="c", subcore_axis_name="s")
    fn = functools.partial(
        pl.kernel,
        mesh=mesh,
        compiler_params=pltpu.CompilerParams(needs_layout_passes=False),
        out_type=(
            jax.ShapeDtypeStruct((EMBED_DIM, BATCH), jnp.float32),
            jax.ShapeDtypeStruct((EMBED_DIM, BATCH), jnp.float32),
        ),
        scratch_types=[
            pltpu.VMEM((BATCH,), jnp.int32),
            pltpu.VMEM((NUM_ROWS,), jnp.float32),
            pltpu.VMEM((_CHUNK,), jnp.float32),
            pltpu.VMEM((_CHUNK,), jnp.float32),
            pltpu.SemaphoreType.DMA,
            pltpu.SemaphoreType.DMA,
            pltpu.SemaphoreType.DMA,
            pltpu.SemaphoreType.DMA,
        ],
    )(_emb_kernel)
    out_ut, out_it = fn(user_embedding.T, item_embedding.T,
                        user_ids.astype(jnp.int32), item_ids.astype(jnp.int32))
    return (out_ut.T, out_it.T)
